# Initial kernel scaffold; baseline (speedup 1.0000x reference)
import jax
import jax.numpy as jnp
from jax.experimental import pallas as pl


def kernel(input_x, table):
    raise NotImplementedError("write your pallas kernel here")


# speedup vs baseline: 1.8644x; 1.8644x over previous
import functools

import jax
import jax.numpy as jnp
from jax import lax
from jax.experimental import pallas as pl
from jax.experimental.pallas import tpu as pltpu
from jax.experimental.pallas import tpu_sc as plsc

BATCH = 16384
HIST = 50
EMBED = 64
VOCAB = 1000000

NC = 2
NS = 16
NW = NC * NS

B = BATCH * HIST
BPW = B // NW
CHUNK = 128
GROUP = 512
CPG = GROUP // CHUNK
NGROUP = BPW // GROUP

_ROW_BYTES = EMBED * 4


def _body(table_hbm, idx_hbm, out_hbm, idx_v, buf0, buf1,
          gsem0, gsem1, osem0, osem1):
    wid = lax.axis_index("s") * NC + lax.axis_index("c")
    base = wid * BPW

    pltpu.sync_copy(idx_hbm.at[wid], idx_v)

    def start_gathers(g, buf, sem):
        for c in range(CPG):
            pltpu.async_copy(
                table_hbm.at[idx_v.at[g * CPG + c]],
                buf.at[pl.ds(c * CHUNK, CHUNK)],
                sem,
            )

    def wait_gathers(buf, sem):
        for c in range(CPG):
            pltpu.make_async_copy(
                table_hbm.at[idx_v.at[c]],
                buf.at[pl.ds(c * CHUNK, CHUNK)],
                sem,
            ).wait()

    def start_out(g, buf, sem):
        pltpu.async_copy(buf, out_hbm.at[pl.ds(base + g * GROUP, GROUP)], sem)

    def wait_out(buf, sem):
        pltpu.make_async_copy(
            buf, out_hbm.at[pl.ds(base, GROUP)], sem
        ).wait()

    start_gathers(0, buf0, gsem0)
    start_gathers(1, buf1, gsem1)

    @pl.loop(0, NGROUP - 2, step=2)
    def _(g):
        wait_gathers(buf0, gsem0)
        start_out(g, buf0, osem0)
        wait_gathers(buf1, gsem1)
        start_out(g + 1, buf1, osem1)
        wait_out(buf0, osem0)
        start_gathers(g + 2, buf0, gsem0)
        wait_out(buf1, osem1)
        start_gathers(g + 3, buf1, gsem1)

    g_last = NGROUP - 2
    wait_gathers(buf0, gsem0)
    start_out(g_last, buf0, osem0)
    wait_gathers(buf1, gsem1)
    start_out(g_last + 1, buf1, osem1)
    wait_out(buf0, osem0)
    wait_out(buf1, osem1)


@jax.jit
def _lookup(table, idx):
    mesh = plsc.VectorSubcoreMesh(core_axis_name="c", subcore_axis_name="s")
    f = pl.kernel(
        _body,
        out_type=jax.ShapeDtypeStruct((B, EMBED), jnp.float32),
        mesh=mesh,
        compiler_params=pltpu.CompilerParams(use_tc_tiling_on_sc=False),
        scratch_types=[
            pltpu.VMEM((NGROUP * CPG, CHUNK), jnp.int32),
            pltpu.VMEM((GROUP, EMBED), jnp.float32),
            pltpu.VMEM((GROUP, EMBED), jnp.float32),
            pltpu.SemaphoreType.DMA,
            pltpu.SemaphoreType.DMA,
            pltpu.SemaphoreType.DMA,
            pltpu.SemaphoreType.DMA,
        ],
    )
    return f(table, idx)


def kernel(input_x, table):
    idx = input_x.reshape(NW, NGROUP * CPG, CHUNK).astype(jnp.int32)
    out = _lookup(table, idx)
    return out.reshape(BATCH, HIST, EMBED)


# speedup vs baseline: 1.8663x; 1.0010x over previous
import functools

import jax
import jax.numpy as jnp
from jax import lax
from jax.experimental import pallas as pl
from jax.experimental.pallas import tpu as pltpu
from jax.experimental.pallas import tpu_sc as plsc

BATCH = 16384
HIST = 50
EMBED = 64
VOCAB = 1000000

NC = 2
NS = 16
NW = NC * NS

B = BATCH * HIST
BPW = B // NW
CHUNK = 512
GROUP = 512
CPG = GROUP // CHUNK
NGROUP = BPW // GROUP

_ROW_BYTES = EMBED * 4


def _body(table_hbm, idx_hbm, out_hbm, idx_v, buf0, buf1,
          gsem0, gsem1, osem0, osem1):
    wid = lax.axis_index("s") * NC + lax.axis_index("c")
    base = wid * BPW

    pltpu.sync_copy(idx_hbm.at[wid], idx_v)

    def start_gathers(g, buf, sem):
        for c in range(CPG):
            pltpu.async_copy(
                table_hbm.at[idx_v.at[g * CPG + c]],
                buf.at[pl.ds(c * CHUNK, CHUNK)],
                sem,
            )

    def wait_gathers(buf, sem):
        for c in range(CPG):
            pltpu.make_async_copy(
                table_hbm.at[idx_v.at[c]],
                buf.at[pl.ds(c * CHUNK, CHUNK)],
                sem,
            ).wait()

    def start_out(g, buf, sem):
        pltpu.async_copy(buf, out_hbm.at[pl.ds(base + g * GROUP, GROUP)], sem)

    def wait_out(buf, sem):
        pltpu.make_async_copy(
            buf, out_hbm.at[pl.ds(base, GROUP)], sem
        ).wait()

    start_gathers(0, buf0, gsem0)
    start_gathers(1, buf1, gsem1)

    @pl.loop(0, NGROUP - 2, step=2)
    def _(g):
        wait_gathers(buf0, gsem0)
        start_out(g, buf0, osem0)
        wait_gathers(buf1, gsem1)
        start_out(g + 1, buf1, osem1)
        wait_out(buf0, osem0)
        start_gathers(g + 2, buf0, gsem0)
        wait_out(buf1, osem1)
        start_gathers(g + 3, buf1, gsem1)

    g_last = NGROUP - 2
    wait_gathers(buf0, gsem0)
    start_out(g_last, buf0, osem0)
    wait_gathers(buf1, gsem1)
    start_out(g_last + 1, buf1, osem1)
    wait_out(buf0, osem0)
    wait_out(buf1, osem1)


@jax.jit
def _lookup(table, idx):
    mesh = plsc.VectorSubcoreMesh(core_axis_name="c", subcore_axis_name="s")
    f = pl.kernel(
        _body,
        out_type=jax.ShapeDtypeStruct((B, EMBED), jnp.float32),
        mesh=mesh,
        compiler_params=pltpu.CompilerParams(use_tc_tiling_on_sc=False),
        scratch_types=[
            pltpu.VMEM((NGROUP * CPG, CHUNK), jnp.int32),
            pltpu.VMEM((GROUP, EMBED), jnp.float32),
            pltpu.VMEM((GROUP, EMBED), jnp.float32),
            pltpu.SemaphoreType.DMA,
            pltpu.SemaphoreType.DMA,
            pltpu.SemaphoreType.DMA,
            pltpu.SemaphoreType.DMA,
        ],
    )
    return f(table, idx)


def kernel(input_x, table):
    idx = input_x.reshape(NW, NGROUP * CPG, CHUNK).astype(jnp.int32)
    out = _lookup(table, idx)
    return out.reshape(BATCH, HIST, EMBED)


# speedup vs baseline: 1.8678x; 1.0008x over previous
import jax
import jax.numpy as jnp
from jax import lax
from jax.experimental import pallas as pl
from jax.experimental.pallas import tpu as pltpu
from jax.experimental.pallas import tpu_sc as plsc

BATCH = 16384
HIST = 50
EMBED = 64

NC = 2
NS = 16
NW = NC * NS

B = BATCH * HIST
BPW = B // NW
GROUP = 256
NGROUP = BPW // GROUP
NBUF = 4

assert (NGROUP - NBUF) % NBUF == 0


def _body(table_hbm, idx_hbm, out_hbm, idx_v, *rest):
    bufs = rest[:NBUF]
    gsems = rest[NBUF:2 * NBUF]
    osems = rest[2 * NBUF:3 * NBUF]

    wid = lax.axis_index("s") * NC + lax.axis_index("c")
    base = wid * BPW

    pltpu.sync_copy(idx_hbm.at[wid], idx_v)

    def start_gather(g, b):
        pltpu.async_copy(table_hbm.at[idx_v.at[g]], bufs[b], gsems[b])

    def wait_gather(b):
        pltpu.make_async_copy(table_hbm.at[idx_v.at[0]], bufs[b],
                              gsems[b]).wait()

    def start_out(g, b):
        pltpu.async_copy(bufs[b], out_hbm.at[pl.ds(base + g * GROUP, GROUP)],
                         osems[b])

    def wait_out(b):
        pltpu.make_async_copy(bufs[b], out_hbm.at[pl.ds(base, GROUP)],
                              osems[b]).wait()

    for b in range(NBUF):
        start_gather(b, b)

    @pl.loop(0, NGROUP - NBUF, step=NBUF)
    def _(g):
        for b in range(NBUF):
            wait_gather(b)
            start_out(g + b, b)
        for b in range(NBUF):
            wait_out(b)
            start_gather(g + NBUF + b, b)

    for b in range(NBUF):
        wait_gather(b)
        start_out(NGROUP - NBUF + b, b)
    for b in range(NBUF):
        wait_out(b)


@jax.jit
def _lookup(table, idx):
    mesh = plsc.VectorSubcoreMesh(core_axis_name="c", subcore_axis_name="s")
    f = pl.kernel(
        _body,
        out_type=jax.ShapeDtypeStruct((B, EMBED), jnp.float32),
        mesh=mesh,
        compiler_params=pltpu.CompilerParams(use_tc_tiling_on_sc=False),
        scratch_types=(
            [pltpu.VMEM((NGROUP, GROUP), jnp.int32)]
            + [pltpu.VMEM((GROUP, EMBED), jnp.float32)] * NBUF
            + [pltpu.SemaphoreType.DMA] * (2 * NBUF)
        ),
    )
    return f(table, idx)


def kernel(input_x, table):
    idx = input_x.reshape(NW, NGROUP, GROUP).astype(jnp.int32)
    out = _lookup(table, idx)
    return out.reshape(BATCH, HIST, EMBED)
